# baseline (speedup 1.0000x reference)
import jax
import jax.numpy as jnp
from jax import lax
from jax.experimental import pallas as pl
from jax.experimental.pallas import tpu as pltpu
from jax.experimental.pallas import tpu_sc as plsc

B = 16384
D = 64
NC, NS, L = 2, 16, 16
NW = NC * NS
BPW = B // NW
CHUNK = 128
NCHUNK = BPW // CHUNK


def _rsqrt_nr(t):
    i = lax.bitcast_convert_type(t, jnp.int32)
    i = jnp.int32(0x5F3759DF) - lax.shift_right_logical(i, 1)
    y = lax.bitcast_convert_type(i, jnp.float32)
    half_t = jnp.float32(0.5) * t
    for _ in range(3):
        y = y * (jnp.float32(1.5) - half_t * y * y)
    return y


def _body(idx_hbm, table_hbm, out_hbm, idx_v, rows_v, sem):
    wid = lax.axis_index("s") * NC + lax.axis_index("c")
    base = wid * BPW

    for j in range(NCHUNK):
        pltpu.sync_copy(idx_hbm.at[pl.ds(base + j * CHUNK, CHUNK)], idx_v.at[j])
    copies = [
        pltpu.async_copy(
            table_hbm.at[idx_v.at[j]],
            rows_v.at[pl.ds(j * CHUNK, CHUNK)],
            sem,
        )
        for j in range(NCHUNK)
    ]
    for c in copies:
        c.wait()

    def group(g, carry):
        row_idx = g * L + lax.iota(jnp.int32, L)
        acc = jnp.zeros((L,), jnp.float32)
        for j in range(D):
            col = jnp.full((L,), j, jnp.int32)
            c = plsc.load_gather(rows_v, [row_idx, col])
            acc = acc + c * c
        inv = _rsqrt_nr(jnp.maximum(acc, jnp.float32(1e-24)))
        for j in range(D):
            col = jnp.full((L,), j, jnp.int32)
            c = plsc.load_gather(rows_v, [row_idx, col])
            plsc.store_scatter(rows_v, [row_idx, col], c * inv)
        return carry

    lax.fori_loop(0, BPW // L, group, 0)

    pltpu.sync_copy(rows_v, out_hbm.at[pl.ds(base, BPW)])


def kernel(batch, table):
    idx = batch.astype(jnp.int32)
    mesh = plsc.VectorSubcoreMesh(core_axis_name="c", subcore_axis_name="s")
    f = pl.kernel(
        _body,
        out_type=jax.ShapeDtypeStruct((B, D), jnp.float32),
        mesh=mesh,
        compiler_params=pltpu.CompilerParams(
            needs_layout_passes=False, use_tc_tiling_on_sc=False
        ),
        scratch_types=[
            pltpu.VMEM((NCHUNK, CHUNK), jnp.int32),
            pltpu.VMEM((BPW, D), jnp.float32),
            pltpu.SemaphoreType.DMA,
        ],
    )
    return f(idx, table)


# speedup vs baseline: 1.5368x; 1.5368x over previous
import jax
import jax.numpy as jnp
from jax import lax
from jax.experimental import pallas as pl
from jax.experimental.pallas import tpu as pltpu
from jax.experimental.pallas import tpu_sc as plsc

B = 16384
D = 64
NC, NS, L = 2, 16, 16
NW = NC * NS
BPW = B // NW
PITCH = D + 1
KDMA = 16


def _rsqrt_nr(t):
    i = lax.bitcast_convert_type(t, jnp.int32)
    i = jnp.int32(0x5F3759DF) - lax.shift_right_logical(i, 1)
    y = lax.bitcast_convert_type(i, jnp.float32)
    half_t = jnp.float32(0.5) * t
    for _ in range(3):
        y = y * (jnp.float32(1.5) - half_t * y * y)
    return y


def _body(idx_hbm, tab_hbm, out_hbm, idx_s, blk_v, rows_v, tbuf_v, sem):
    wid = lax.axis_index("s") * NC + lax.axis_index("c")
    base = wid * BPW

    pltpu.sync_copy(idx_hbm.at[pl.ds(base, BPW)], idx_s)

    def chunk(cc, carry):
        i0 = cc * KDMA
        rvec = idx_s[pl.ds(i0, KDMA)]
        copies = []
        for k in range(KDMA):
            r = rvec[k]
            r8 = (r // 8) * 8
            copies.append(
                pltpu.async_copy(
                    tab_hbm.at[pl.ds(r8, 8), :],
                    blk_v.at[k],
                    sem,
                )
            )
        for c in copies:
            c.wait()
        for k in range(KDMA):
            r = rvec[k]
            sub = r - (r // 8) * 8
            for jj in range(D // L):
                rows_v[i0 + k, pl.ds(jj * L, L)] = blk_v[k, sub, pl.ds(jj * L, L)]
        return carry

    lax.fori_loop(0, BPW // KDMA, chunk, 0)

    def group(g, carry):
        row_idx = g * L + lax.iota(jnp.int32, L)
        acc = jnp.zeros((L,), jnp.float32)
        for j in range(D):
            col = jnp.full((L,), j, jnp.int32)
            c = plsc.load_gather(rows_v, [row_idx, col])
            acc = acc + c * c
        inv = _rsqrt_nr(jnp.maximum(acc, jnp.float32(1e-24)))
        for j in range(D):
            col = jnp.full((L,), j, jnp.int32)
            c = plsc.load_gather(rows_v, [row_idx, col])
            tbuf_v[j, pl.ds(g * L, L)] = c * inv
        return carry

    lax.fori_loop(0, BPW // L, group, 0)

    pltpu.sync_copy(tbuf_v, out_hbm.at[:, pl.ds(base, BPW)])


def kernel(batch, table):
    idx = batch.astype(jnp.int32)
    mesh = plsc.VectorSubcoreMesh(core_axis_name="c", subcore_axis_name="s")
    f = pl.kernel(
        _body,
        out_type=jax.ShapeDtypeStruct((D, B), jnp.float32),
        mesh=mesh,
        compiler_params=pltpu.CompilerParams(
            needs_layout_passes=False, use_tc_tiling_on_sc=True
        ),
        scratch_types=[
            pltpu.VMEM((BPW,), jnp.int32),
            pltpu.VMEM((KDMA, 8, D), jnp.float32),
            pltpu.VMEM((BPW, PITCH), jnp.float32),
            pltpu.VMEM((D, BPW), jnp.float32),
            pltpu.SemaphoreType.DMA,
        ],
    )
    return f(idx, table).T


# speedup vs baseline: 2.1881x; 1.4238x over previous
import jax
import jax.numpy as jnp
from jax import lax
from jax.experimental import pallas as pl
from jax.experimental.pallas import tpu as pltpu
from jax.experimental.pallas import tpu_sc as plsc

B = 16384
D = 64
NC, NS, L = 2, 16, 16
NW = NC * NS
BPW = B // NW
PITCH = D + 1
NBUF = 3


def _rsqrt_nr(t):
    i = lax.bitcast_convert_type(t, jnp.int32)
    i = jnp.int32(0x5F3759DF) - lax.shift_right_logical(i, 1)
    y = lax.bitcast_convert_type(i, jnp.float32)
    half_t = jnp.float32(0.5) * t
    for _ in range(3):
        y = y * (jnp.float32(1.5) - half_t * y * y)
    return y


def _body(idx_hbm, tab_hbm, out_hbm, idx_s, slab_v, rows_v, tbuf_v, *sems):
    wid = lax.axis_index("s") * NC + lax.axis_index("c")
    base = wid * BPW

    pltpu.sync_copy(idx_hbm.at[pl.ds(base, BPW)], idx_s.at[pl.ds(0, BPW)])

    def slab_copy(r, b):
        c = (r >> 7) * 128
        return pltpu.async_copy(
            tab_hbm.at[:, pl.ds(c, 128)], slab_v.at[b], sems[b]
        )

    def slab_wait(b):
        pltpu.make_async_copy(
            tab_hbm.at[:, pl.ds(0, 128)], slab_v.at[b], sems[b]
        ).wait()

    rvec0 = idx_s[pl.ds(0, L)]
    for b in range(NBUF):
        slab_copy(rvec0[b], b)

    def extract(r, i, b):
        slab_wait(b)
        col = r - (r >> 7) * 128
        colv = jnp.full((L,), col, jnp.int32)
        bv = jnp.full((L,), b, jnp.int32)
        for m in range(D // L):
            jv = m * L + lax.iota(jnp.int32, L)
            v = plsc.load_gather(slab_v, [bv, jv, colv])
            rows_v[i, pl.ds(m * L, L)] = v

    NFULL = BPW // NBUF

    def chunk(cc, carry):
        i0 = cc * NBUF
        rvec = idx_s[pl.ds(i0, L)]
        for b in range(NBUF):
            i = i0 + b
            extract(rvec[b], i, b)

            @pl.when(i + NBUF < BPW)
            def _():
                slab_copy(rvec[b + NBUF], b)

        return carry

    lax.fori_loop(0, NFULL, chunk, 0)
    rvec_t = idx_s[pl.ds(NFULL * NBUF, L)]
    for b in range(BPW - NFULL * NBUF):
        extract(rvec_t[b], NFULL * NBUF + b, b)

    def group(g, carry):
        row_idx = g * L + lax.iota(jnp.int32, L)
        acc = jnp.zeros((L,), jnp.float32)
        for j in range(D):
            col = jnp.full((L,), j, jnp.int32)
            c = plsc.load_gather(rows_v, [row_idx, col])
            acc = acc + c * c
        inv = _rsqrt_nr(jnp.maximum(acc, jnp.float32(1e-24)))
        for j in range(D):
            col = jnp.full((L,), j, jnp.int32)
            c = plsc.load_gather(rows_v, [row_idx, col])
            tbuf_v[j, pl.ds(g * L, L)] = c * inv
        return carry

    lax.fori_loop(0, BPW // L, group, 0)

    pltpu.sync_copy(tbuf_v, out_hbm.at[:, pl.ds(base, BPW)])


def kernel(batch, table):
    idx = batch.astype(jnp.int32)
    tab_t = table.T
    mesh = plsc.VectorSubcoreMesh(core_axis_name="c", subcore_axis_name="s")
    f = pl.kernel(
        _body,
        out_type=jax.ShapeDtypeStruct((D, B), jnp.float32),
        mesh=mesh,
        compiler_params=pltpu.CompilerParams(
            needs_layout_passes=False, use_tc_tiling_on_sc=True
        ),
        scratch_types=[
            pltpu.VMEM((BPW + L,), jnp.int32),
            pltpu.VMEM((NBUF, D, 128), jnp.float32),
            pltpu.VMEM((BPW, PITCH), jnp.float32),
            pltpu.VMEM((D, BPW), jnp.float32),
        ]
        + [pltpu.SemaphoreType.DMA] * NBUF,
    )
    return f(idx, tab_t).T


# speedup vs baseline: 2.7442x; 1.2541x over previous
import jax
import jax.numpy as jnp
from jax import lax
from jax.experimental import pallas as pl
from jax.experimental.pallas import tpu as pltpu
from jax.experimental.pallas import tpu_sc as plsc

B = 16384
D = 64
NC, NS, L = 2, 16, 16
NW = NC * NS
BPW = B // NW
PITCH = D + 1
NBUF = 4


def _rsqrt_nr(t):
    i = lax.bitcast_convert_type(t, jnp.int32)
    i = jnp.int32(0x5F3759DF) - lax.shift_right_logical(i, 1)
    y = lax.bitcast_convert_type(i, jnp.float32)
    half_t = jnp.float32(0.5) * t
    for _ in range(3):
        y = y * (jnp.float32(1.5) - half_t * y * y)
    return y


def _body(idx_hbm, tab_hbm, out_hbm, idx_s, slab_v, rows_v, tbuf_v, *sems):
    wid = lax.axis_index("s") * NC + lax.axis_index("c")
    base = wid * BPW

    pltpu.sync_copy(idx_hbm.at[pl.ds(base, BPW)], idx_s.at[pl.ds(0, BPW)])

    def slab_copy(r, b):
        c = (r >> 7) * 128
        return pltpu.async_copy(
            tab_hbm.at[:, pl.ds(c, 128)], slab_v.at[b], sems[b]
        )

    def slab_wait(b):
        pltpu.make_async_copy(
            tab_hbm.at[:, pl.ds(0, 128)], slab_v.at[b], sems[b]
        ).wait()

    rvec0 = idx_s[pl.ds(0, L)]
    for b in range(NBUF):
        slab_copy(rvec0[b], b)

    def extract(r, i, b):
        slab_wait(b)
        col = r - (r >> 7) * 128
        colv = jnp.full((L,), col, jnp.int32)
        bv = jnp.full((L,), b, jnp.int32)
        for m in range(D // L):
            jv = m * L + lax.iota(jnp.int32, L)
            v = plsc.load_gather(slab_v, [bv, jv, colv])
            rows_v[i, pl.ds(m * L, L)] = v

    def normalize_group(g):
        row_idx = g * L + lax.iota(jnp.int32, L)
        acc = jnp.zeros((L,), jnp.float32)
        for j in range(D):
            col = jnp.full((L,), j, jnp.int32)
            c = plsc.load_gather(rows_v, [row_idx, col])
            acc = acc + c * c
        inv = _rsqrt_nr(jnp.maximum(acc, jnp.float32(1e-24)))
        slot = (g - (g >> 3) * 8) * L
        for j in range(D):
            col = jnp.full((L,), j, jnp.int32)
            c = plsc.load_gather(rows_v, [row_idx, col])
            tbuf_v[j, pl.ds(slot, L)] = c * inv

    def chunk(cc, carry):
        i0 = cc * NBUF
        rvec = idx_s[pl.ds(i0, L)]
        for b in range(NBUF):
            i = i0 + b
            extract(rvec[b], i, b)

            @pl.when(i + NBUF < BPW)
            def _():
                slab_copy(rvec[b + NBUF], b)

        @pl.when((cc & 3) == 3)
        def _():
            g = cc >> 2
            normalize_group(g)

            @pl.when((g & 7) == 7)
            def _():
                p = g >> 3
                pltpu.sync_copy(
                    tbuf_v, out_hbm.at[:, pl.ds(base + p * 128, 128)]
                )

        return carry

    lax.fori_loop(0, BPW // NBUF, chunk, 0)


def kernel(batch, table):
    idx = batch.astype(jnp.int32)
    tab_t = table.T
    mesh = plsc.VectorSubcoreMesh(core_axis_name="c", subcore_axis_name="s")
    f = pl.kernel(
        _body,
        out_type=jax.ShapeDtypeStruct((D, B), jnp.float32),
        mesh=mesh,
        compiler_params=pltpu.CompilerParams(
            needs_layout_passes=False, use_tc_tiling_on_sc=True
        ),
        scratch_types=[
            pltpu.VMEM((BPW + L,), jnp.int32),
            pltpu.VMEM((NBUF, D, 128), jnp.float32),
            pltpu.VMEM((BPW, PITCH), jnp.float32),
            pltpu.VMEM((D, 128), jnp.float32),
        ]
        + [pltpu.SemaphoreType.DMA] * NBUF,
    )
    return f(idx, tab_t).T


# speedup vs baseline: 3.2860x; 1.1974x over previous
import jax
import jax.numpy as jnp
from jax import lax
from jax.experimental import pallas as pl
from jax.experimental.pallas import tpu as pltpu
from jax.experimental.pallas import tpu_sc as plsc

B = 16384
D = 64
NC, NS, L = 2, 16, 16
NW = NC * NS
BPW = B // NW
PITCH = D + 1
NBUF = 8


def _rsqrt_nr(t):
    i = lax.bitcast_convert_type(t, jnp.int32)
    i = jnp.int32(0x5F3759DF) - lax.shift_right_logical(i, 1)
    y = lax.bitcast_convert_type(i, jnp.float32)
    half_t = jnp.float32(0.5) * t
    for _ in range(3):
        y = y * (jnp.float32(1.5) - half_t * y * y)
    return y


def _body(idx_hbm, tab_hbm, out_hbm, idx_s, slab_v, rows_v, tbuf_v, *sems):
    wid = lax.axis_index("s") * NC + lax.axis_index("c")
    base = wid * BPW

    pltpu.sync_copy(idx_hbm.at[pl.ds(base, BPW)], idx_s.at[pl.ds(0, BPW)])

    def slab_copy(r, b):
        c = (r >> 7) * 128
        return pltpu.async_copy(
            tab_hbm.at[:, pl.ds(c, 128)], slab_v.at[b], sems[b]
        )

    def slab_wait(b):
        pltpu.make_async_copy(
            tab_hbm.at[:, pl.ds(0, 128)], slab_v.at[b], sems[b]
        ).wait()

    rvec0 = idx_s[pl.ds(0, L)]
    for b in range(NBUF):
        slab_copy(rvec0[b], b)

    def extract(r, i, b):
        slab_wait(b)
        col = r - (r >> 7) * 128
        colv = jnp.full((L,), col, jnp.int32)
        bv = jnp.full((L,), b, jnp.int32)
        islot = i - (i >> 5) * 32
        for m in range(D // L):
            jv = m * L + lax.iota(jnp.int32, L)
            v = plsc.load_gather(slab_v, [bv, jv, colv])
            rows_v[islot, pl.ds(m * L, L)] = v

    def normalize_group(g):
        row_idx = (g - (g >> 1) * 2) * L + lax.iota(jnp.int32, L)
        acc = jnp.zeros((L,), jnp.float32)
        for j in range(D):
            col = jnp.full((L,), j, jnp.int32)
            c = plsc.load_gather(rows_v, [row_idx, col])
            acc = acc + c * c
        inv = _rsqrt_nr(jnp.maximum(acc, jnp.float32(1e-24)))
        slot = (g - (g >> 3) * 8) * L
        for j in range(D):
            col = jnp.full((L,), j, jnp.int32)
            c = plsc.load_gather(rows_v, [row_idx, col])
            tbuf_v[j, pl.ds(slot, L)] = c * inv

    def chunk(cc, carry):
        i0 = cc * NBUF
        rvec = idx_s[pl.ds(i0, L)]
        for b in range(NBUF):
            i = i0 + b
            extract(rvec[b], i, b)

            @pl.when(i + NBUF < BPW)
            def _():
                slab_copy(rvec[b + NBUF], b)

        @pl.when((cc & 1) == 1)
        def _():
            g = cc >> 1
            normalize_group(g)

            @pl.when((g & 7) == 7)
            def _():
                p = g >> 3
                pltpu.sync_copy(
                    tbuf_v, out_hbm.at[:, pl.ds(base + p * 128, 128)]
                )

        return carry

    lax.fori_loop(0, BPW // NBUF, chunk, 0)


def kernel(batch, table):
    idx = batch.astype(jnp.int32)
    tab_t = table.T
    mesh = plsc.VectorSubcoreMesh(core_axis_name="c", subcore_axis_name="s")
    f = pl.kernel(
        _body,
        out_type=jax.ShapeDtypeStruct((D, B), jnp.float32),
        mesh=mesh,
        compiler_params=pltpu.CompilerParams(
            needs_layout_passes=False, use_tc_tiling_on_sc=True
        ),
        scratch_types=[
            pltpu.VMEM((BPW + L,), jnp.int32),
            pltpu.VMEM((NBUF, D, 128), jnp.float32),
            pltpu.VMEM((2 * L, PITCH), jnp.float32),
            pltpu.VMEM((D, 128), jnp.float32),
        ]
        + [pltpu.SemaphoreType.DMA] * NBUF,
    )
    return f(idx, tab_t).T
